# baseline (speedup 1.0000x reference)
import functools

import jax
import jax.numpy as jnp
from jax import lax
from jax.experimental import pallas as pl
from jax.experimental.pallas import tpu as pltpu
from jax.experimental.pallas import tpu_sc as plsc

N_NODES = 100000
N_GRAPHS = 64
E = 1600000
H = 16

LANES = 128
E_ROWS = E // LANES
CR = 8
NCH = E_ROWS // CR
TAIL_R = E_ROWS - NCH * CR
CE = CR * LANES
NSC = 16
DCH = 1000
NDC = N_NODES // DCH
CW = CE // 8
EW = E // 8
TW = TAIL_R * LANES // 8

_mesh = plsc.VectorSubcoreMesh(core_axis_name="c", subcore_axis_name="s")


def _fill_rows(ref, n_rows, vec):
    def body(i, carry):
        ref[i] = vec
        return carry
    lax.fori_loop(0, n_rows, body, 0)


def _strided(start, stride, total, body):
    n_my = (total - start + stride - 1) // stride

    def loop_body(i, carry):
        body(start + i * stride)
        return carry

    lax.fori_loop(0, n_my, loop_body, 0)


def _zero_acc(acc, sid, zbuf):
    _fill_rows(zbuf, DCH, jnp.zeros((H,), jnp.float32))
    _strided(sid, NSC, NDC,
             lambda c: pltpu.sync_copy(zbuf.at[pl.ds(0, DCH)],
                                       acc.at[pl.ds(c * DCH, DCH)]))
    plsc.subcore_barrier()


def _dump_acc(acc, cid, sid, out):
    plsc.subcore_barrier()
    _strided(sid, NSC, NDC,
             lambda c: pltpu.sync_copy(acc.at[pl.ds(c * DCH, DCH)],
                                       out.at[cid, pl.ds(c * DCH, DCH)]))


@functools.partial(
    pl.kernel,
    out_type=jax.ShapeDtypeStruct((2, N_NODES, H), jnp.float32),
    mesh=_mesh,
    compiler_params=pltpu.CompilerParams(use_tc_tiling_on_sc=False),
    scratch_types=[
        pltpu.VMEM_SHARED((N_NODES, H), jnp.float32),
        pltpu.VMEM((CR, LANES), jnp.int32),
        pltpu.VMEM((CE, H), jnp.float32),
    ],
)
def _sc_count(eidx, out, acc, idxb, ones):
    cid = lax.axis_index("c")
    sid = lax.axis_index("s")
    _zero_acc(acc, sid, ones)
    _fill_rows(ones, CE, jnp.ones((H,), jnp.float32))
    plsc.subcore_barrier()

    def chunk(c):
        pltpu.sync_copy(eidx.at[cid, pl.ds(c * CR, CR)], idxb)
        for j in range(CR):
            pltpu.sync_copy(ones.at[pl.ds(j * LANES, LANES)],
                            acc.at[idxb.at[j]], add=True)

    _strided(sid, NSC, NCH, chunk)

    @pl.when(sid == NSC - 1)
    def _():
        pltpu.sync_copy(eidx.at[cid, pl.ds(NCH * CR, TAIL_R)],
                        idxb.at[pl.ds(0, TAIL_R)])
        for j in range(TAIL_R):
            pltpu.sync_copy(ones.at[pl.ds(j * LANES, LANES)],
                            acc.at[idxb.at[j]], add=True)

    _dump_acc(acc, cid, sid, out)


@functools.partial(
    pl.kernel,
    out_type=jax.ShapeDtypeStruct((2, N_NODES, H), jnp.float32),
    mesh=_mesh,
    compiler_params=pltpu.CompilerParams(use_tc_tiling_on_sc=False),
    scratch_types=[
        pltpu.VMEM_SHARED((N_NODES, H), jnp.float32),
        pltpu.VMEM((CR, LANES), jnp.int32),
        pltpu.VMEM((CE, H), jnp.float32),
    ],
)
def _sc_scatter(eidx, vals, out, acc, idxb, vbuf):
    cid = lax.axis_index("c")
    sid = lax.axis_index("s")
    _zero_acc(acc, sid, vbuf)

    def chunk(c):
        pltpu.sync_copy(eidx.at[cid, pl.ds(c * CR, CR)], idxb)
        pltpu.sync_copy(vals.at[pl.ds(c * CE, CE)], vbuf)
        for j in range(CR):
            pltpu.sync_copy(vbuf.at[pl.ds(j * LANES, LANES)],
                            acc.at[idxb.at[j]], add=True)

    _strided(sid, NSC, NCH, chunk)

    @pl.when(sid == NSC - 1)
    def _():
        pltpu.sync_copy(eidx.at[cid, pl.ds(NCH * CR, TAIL_R)],
                        idxb.at[pl.ds(0, TAIL_R)])
        pltpu.sync_copy(vals.at[pl.ds(NCH * CE, TAIL_R * LANES)],
                        vbuf.at[pl.ds(0, TAIL_R * LANES)])
        for j in range(TAIL_R):
            pltpu.sync_copy(vbuf.at[pl.ds(j * LANES, LANES)],
                            acc.at[idxb.at[j]], add=True)

    _dump_acc(acc, cid, sid, out)


@functools.partial(
    pl.kernel,
    out_type=[jax.ShapeDtypeStruct((E, H), jnp.float32),
              jax.ShapeDtypeStruct((E, H), jnp.float32)],
    mesh=_mesh,
    compiler_params=pltpu.CompilerParams(use_tc_tiling_on_sc=False),
    scratch_types=[
        pltpu.VMEM_SHARED((N_NODES, H), jnp.float32),
        pltpu.VMEM((CR, LANES), jnp.int32),
        pltpu.VMEM((CE, H), jnp.float32),
    ],
)
def _sc_gather(eidx, a_tab, b_tab, ga, gb, tab_s, idxb, buf):
    cid = lax.axis_index("c")
    sid = lax.axis_index("s")

    def prefetch(tab):
        _strided(sid, NSC, NDC,
                 lambda c: pltpu.sync_copy(tab.at[pl.ds(c * DCH, DCH)],
                                           tab_s.at[pl.ds(c * DCH, DCH)]))

    def do_rows(out, c, n_rows):
        pltpu.sync_copy(eidx.at[cid, pl.ds(c * CR, n_rows)],
                        idxb.at[pl.ds(0, n_rows)])
        for j in range(n_rows):
            pltpu.sync_copy(tab_s.at[idxb.at[j]],
                            buf.at[pl.ds(j * LANES, LANES)])
        pltpu.sync_copy(buf.at[pl.ds(0, n_rows * LANES)],
                        out.at[pl.ds(c * CE, n_rows * LANES)])

    def run(tab, out):
        prefetch(tab)
        plsc.subcore_barrier()
        _strided(sid, NSC, NCH, lambda c: do_rows(out, c, CR))

        @pl.when(sid == NSC - 1)
        def _():
            do_rows(out, NCH, TAIL_R)

    @pl.when(cid == 0)
    def _():
        run(a_tab, ga)

    @pl.when(cid == 1)
    def _():
        run(b_tab, gb)


BN = 1000
NB = N_NODES // BN
BE = 2000
NEB = (E // 8) // BE


def _graph_body(s_ref, c_ref, batch_ref, gsum_ref, gcnt_ref):
    i = pl.program_id(0)
    bt = batch_ref[0, 0, :]
    onehot = (bt[:, None] == lax.broadcasted_iota(jnp.int32, (BN, N_GRAPHS), 1)
              ).astype(jnp.float32)
    dims = (((0,), (0,)), ((), ()))
    gs = lax.dot_general(onehot, s_ref[...], dims,
                         preferred_element_type=jnp.float32)
    gc = lax.dot_general(onehot, c_ref[...], dims,
                         preferred_element_type=jnp.float32)

    @pl.when(i == 0)
    def _():
        gsum_ref[...] = jnp.zeros_like(gsum_ref)
        gcnt_ref[...] = jnp.zeros_like(gcnt_ref)

    gsum_ref[...] += gs
    gcnt_ref[...] += gc


def _tc_graph(S, cnt, batch3):
    return pl.pallas_call(
        _graph_body,
        grid=(NB,),
        in_specs=[
            pl.BlockSpec((BN, H), lambda i: (i, 0)),
            pl.BlockSpec((BN, H), lambda i: (i, 0)),
            pl.BlockSpec((1, 1, BN), lambda i: (i, 0, 0)),
        ],
        out_specs=[
            pl.BlockSpec((N_GRAPHS, H), lambda i: (0, 0)),
            pl.BlockSpec((N_GRAPHS, H), lambda i: (0, 0)),
        ],
        out_shape=[
            jax.ShapeDtypeStruct((N_GRAPHS, H), jnp.float32),
            jax.ShapeDtypeStruct((N_GRAPHS, H), jnp.float32),
        ],
    )(S, cnt, batch3)


def _tables_body(ss_ref, sd_ref, cs_ref, cd_ref, batch_ref, gsum_ref, gcnt_ref,
                 w_ref, b_ref, a_ref, b_out_ref):
    g = gsum_ref[...] / jnp.maximum(gcnt_ref[...], 1.0)
    C = jnp.dot(g, w_ref[5], preferred_element_type=jnp.float32) + b_ref[...]
    m_s = ss_ref[...] / jnp.maximum(cs_ref[...], 1.0)
    m_d = sd_ref[...] / jnp.maximum(cd_ref[...], 1.0)
    bt = batch_ref[0, 0, :]
    onehot = (bt[:, None] == lax.broadcasted_iota(jnp.int32, (BN, N_GRAPHS), 1)
              ).astype(jnp.float32)
    a_ref[...] = (jnp.dot(m_s, w_ref[1], preferred_element_type=jnp.float32)
                  + jnp.dot(m_d, w_ref[3], preferred_element_type=jnp.float32)
                  + jnp.dot(onehot, C, preferred_element_type=jnp.float32))
    b_out_ref[...] = (jnp.dot(m_s, w_ref[2], preferred_element_type=jnp.float32)
                      + jnp.dot(m_d, w_ref[4], preferred_element_type=jnp.float32))


def _tc_tables(Ss, Sd, cs, cd, batch3, gsum, gcnt, W, b):
    return pl.pallas_call(
        _tables_body,
        grid=(NB,),
        in_specs=[
            pl.BlockSpec((BN, H), lambda i: (i, 0)),
            pl.BlockSpec((BN, H), lambda i: (i, 0)),
            pl.BlockSpec((BN, H), lambda i: (i, 0)),
            pl.BlockSpec((BN, H), lambda i: (i, 0)),
            pl.BlockSpec((1, 1, BN), lambda i: (i, 0, 0)),
            pl.BlockSpec((N_GRAPHS, H), lambda i: (0, 0)),
            pl.BlockSpec((N_GRAPHS, H), lambda i: (0, 0)),
            pl.BlockSpec((6, H, H), lambda i: (0, 0, 0)),
            pl.BlockSpec((1, H), lambda i: (0, 0)),
        ],
        out_specs=[
            pl.BlockSpec((BN, H), lambda i: (i, 0)),
            pl.BlockSpec((BN, H), lambda i: (i, 0)),
        ],
        out_shape=[
            jax.ShapeDtypeStruct((N_NODES, H), jnp.float32),
            jax.ShapeDtypeStruct((N_NODES, H), jnp.float32),
        ],
    )(Ss, Sd, cs, cd, batch3, gsum, gcnt, W, b)


NR32 = E // 32
BR = 1000
NRB = NR32 // BR


def _in_body(in32_ref, mcat_ref, pcat_ref, xw_ref, in16_ref):
    v = in32_ref[...]
    xw_ref[...] = jnp.dot(v, mcat_ref[...], preferred_element_type=jnp.float32)
    in16_ref[...] = jnp.dot(v, pcat_ref[...],
                            preferred_element_type=jnp.float32)


def _tc_in(in32, Mcat, Pcat):
    return pl.pallas_call(
        _in_body,
        grid=(NRB,),
        in_specs=[
            pl.BlockSpec((BR, LANES), lambda i: (i, 0)),
            pl.BlockSpec((LANES, 4 * LANES), lambda i: (0, 0)),
            pl.BlockSpec((LANES, 4 * LANES), lambda i: (0, 0)),
        ],
        out_specs=[
            pl.BlockSpec((BR, 4 * LANES), lambda i: (i, 0)),
            pl.BlockSpec((BR, 4 * LANES), lambda i: (i, 0)),
        ],
        out_shape=[
            jax.ShapeDtypeStruct((NR32, 4 * LANES), jnp.float32),
            jax.ShapeDtypeStruct((NR32, 4 * LANES), jnp.float32),
        ],
    )(in32, Mcat, Pcat)


def _in_weights(W0):
    def cat(w):
        M = jnp.zeros((LANES, 4 * LANES), jnp.float32)
        for q in range(4):
            for a in range(8):
                r0 = 4 * (8 * q + a)
                c0 = 128 * q + 16 * a
                M = M.at[r0:r0 + 4, c0:c0 + H].set(w)
        return M
    return cat(W0), cat(jnp.eye(4, H, dtype=jnp.float32))


def _edge0_body(xw_ref, ga_ref, gb_ref, w1_ref, b1_ref, x0_ref, h1_ref):
    x0 = xw_ref[...] + ga_ref[...] + gb_ref[...]
    x0_ref[...] = x0
    h1_ref[...] = jax.nn.gelu(
        jnp.dot(x0, w1_ref[...], preferred_element_type=jnp.float32)
        + b1_ref[...])


def _tc_edge0(xw128, Ga128, Gb128, W1d, b1t):
    return pl.pallas_call(
        _edge0_body,
        grid=(NEB,),
        in_specs=[
            pl.BlockSpec((BE, LANES), lambda i: (i, 0)),
            pl.BlockSpec((BE, LANES), lambda i: (i, 0)),
            pl.BlockSpec((BE, LANES), lambda i: (i, 0)),
            pl.BlockSpec((LANES, LANES), lambda i: (0, 0)),
            pl.BlockSpec((1, LANES), lambda i: (0, 0)),
        ],
        out_specs=[
            pl.BlockSpec((BE, LANES), lambda i: (i, 0)),
            pl.BlockSpec((BE, LANES), lambda i: (i, 0)),
        ],
        out_shape=[
            jax.ShapeDtypeStruct((E // 8, LANES), jnp.float32),
            jax.ShapeDtypeStruct((E // 8, LANES), jnp.float32),
        ],
    )(xw128, Ga128, Gb128, W1d, b1t)


def _edge_mid_body(x_ref, h_ref, ga_ref, gb_ref, w20_ref, w1n_ref, b1n_ref,
                   x_out_ref, h_out_ref):
    xn = x_ref[...] + jax.nn.gelu(
        jnp.dot(h_ref[...], w20_ref[...], preferred_element_type=jnp.float32)
        + ga_ref[...] + gb_ref[...])
    x_out_ref[...] = xn
    h_out_ref[...] = jax.nn.gelu(
        jnp.dot(xn, w1n_ref[...], preferred_element_type=jnp.float32)
        + b1n_ref[...])


def _tc_edge_mid(x128, h128, Ga128, Gb128, W20d, W1nd, b1nt):
    return pl.pallas_call(
        _edge_mid_body,
        grid=(NEB,),
        in_specs=[
            pl.BlockSpec((BE, LANES), lambda i: (i, 0)),
            pl.BlockSpec((BE, LANES), lambda i: (i, 0)),
            pl.BlockSpec((BE, LANES), lambda i: (i, 0)),
            pl.BlockSpec((BE, LANES), lambda i: (i, 0)),
            pl.BlockSpec((LANES, LANES), lambda i: (0, 0)),
            pl.BlockSpec((LANES, LANES), lambda i: (0, 0)),
            pl.BlockSpec((1, LANES), lambda i: (0, 0)),
        ],
        out_specs=[
            pl.BlockSpec((BE, LANES), lambda i: (i, 0)),
            pl.BlockSpec((BE, LANES), lambda i: (i, 0)),
        ],
        out_shape=[
            jax.ShapeDtypeStruct((E // 8, LANES), jnp.float32),
            jax.ShapeDtypeStruct((E // 8, LANES), jnp.float32),
        ],
    )(x128, h128, Ga128, Gb128, W20d, W1nd, b1nt)


def _edge_last_body(x_ref, h_ref, ga_ref, gb_ref, w20_ref, x_out_ref):
    x_out_ref[...] = x_ref[...] + jax.nn.gelu(
        jnp.dot(h_ref[...], w20_ref[...], preferred_element_type=jnp.float32)
        + ga_ref[...] + gb_ref[...])


def _tc_edge_last(x128, h128, Ga128, Gb128, W20d):
    return pl.pallas_call(
        _edge_last_body,
        grid=(NEB,),
        in_specs=[
            pl.BlockSpec((BE, LANES), lambda i: (i, 0)),
            pl.BlockSpec((BE, LANES), lambda i: (i, 0)),
            pl.BlockSpec((BE, LANES), lambda i: (i, 0)),
            pl.BlockSpec((BE, LANES), lambda i: (i, 0)),
            pl.BlockSpec((LANES, LANES), lambda i: (0, 0)),
        ],
        out_specs=pl.BlockSpec((BE, LANES), lambda i: (i, 0)),
        out_shape=jax.ShapeDtypeStruct((E // 8, LANES), jnp.float32),
    )(x128, h128, Ga128, Gb128, W20d)


def _out_body(gsum_ref, gcnt_ref, w_ref, b_ref, out_ref):
    g = gsum_ref[...] / jnp.maximum(gcnt_ref[...], 1.0)
    out_ref[...] = jnp.dot(g, w_ref[...],
                           preferred_element_type=jnp.float32) + b_ref[...]


def _tc_out(gsum, gcnt, Wpad, bpad):
    return pl.pallas_call(
        _out_body,
        in_specs=[
            pl.BlockSpec((N_GRAPHS, H), lambda: (0, 0)),
            pl.BlockSpec((N_GRAPHS, H), lambda: (0, 0)),
            pl.BlockSpec((H, H), lambda: (0, 0)),
            pl.BlockSpec((1, H), lambda: (0, 0)),
        ],
        out_specs=pl.BlockSpec((N_GRAPHS, H), lambda: (0, 0)),
        out_shape=jax.ShapeDtypeStruct((N_GRAPHS, H), jnp.float32),
    )(gsum, gcnt, Wpad, bpad)


def _bd(W):
    return jnp.kron(jnp.eye(8, dtype=W.dtype), W)


def kernel(in_rank2, edge_index, batch, W_in, b_in, W1_0, b1_0, W2_0, b2_0,
           W1_1, b1_1, W2_1, b2_1, W_out, b_out):
    eidx = edge_index.reshape(2, E_ROWS, LANES)
    batch3 = batch.reshape(NB, 1, BN)
    Mcat, Pcat = _in_weights(W_in[0])
    xw0, in16 = _tc_in(in_rank2.reshape(NR32, LANES), Mcat, Pcat)
    W_in16 = jnp.pad(W_in, ((0, 0), (0, H - W_in.shape[1]), (0, 0)))

    cnts = _sc_count(eidx)
    cs, cd = cnts[0], cnts[1]

    def round_tables(vals16, W6, b):
        S = _sc_scatter(eidx, vals16)
        Ss, Sd = S[0], S[1]
        gsum, gcnt = _tc_graph(Ss, cs, batch3)
        A, B = _tc_tables(Ss, Sd, cs, cd, batch3, gsum, gcnt,
                          W6, b.reshape(1, H))
        Ga, Gb = _sc_gather(eidx, A, B)
        return Ga.reshape(EW, LANES), Gb.reshape(EW, LANES), gcnt

    Ga0, Gb0, gcnt = round_tables(in16.reshape(E, H), W_in16, b_in)
    x0, h1 = _tc_edge0(xw0.reshape(EW, LANES), Ga0, Gb0, _bd(W1_0),
                       jnp.tile(b1_0, 8).reshape(1, LANES))

    Ga1, Gb1, _ = round_tables(h1.reshape(E, H), W2_0, b2_0)
    x1, h2 = _tc_edge_mid(x0, h1, Ga1, Gb1, _bd(W2_0[0]), _bd(W1_1),
                          jnp.tile(b1_1, 8).reshape(1, LANES))

    Ga2, Gb2, _ = round_tables(h2.reshape(E, H), W2_1, b2_1)
    x2 = _tc_edge_last(x1, h2, Ga2, Gb2, _bd(W2_1[0]))

    S3 = _sc_scatter(eidx, x2.reshape(E, H))
    gsum3, _ = _tc_graph(S3[0], cs, batch3)
    Wo = jnp.pad(W_out, ((0, 0), (0, H - W_out.shape[1])))
    bo = jnp.pad(b_out, (0, H - b_out.shape[0])).reshape(1, H)
    out_full = _tc_out(gsum3, gcnt, Wo, bo)
    return out_full[:, :1]


# speedup vs baseline: 1.0428x; 1.0428x over previous
import functools

import jax
import jax.numpy as jnp
from jax import lax
from jax.experimental import pallas as pl
from jax.experimental.pallas import tpu as pltpu
from jax.experimental.pallas import tpu_sc as plsc

N_NODES = 100000
N_GRAPHS = 64
E = 1600000
H = 16

LANES = 128
E_ROWS = E // LANES
CR = 8
NCH = E_ROWS // CR
TAIL_R = E_ROWS - NCH * CR
CE = CR * LANES
NSC = 16
DCH = 1000
NDC = N_NODES // DCH
CW = CE // 8
EW = E // 8
TW = TAIL_R * LANES // 8

_mesh = plsc.VectorSubcoreMesh(core_axis_name="c", subcore_axis_name="s")


def _fill_rows(ref, n_rows, vec):
    def body(i, carry):
        ref[i] = vec
        return carry
    lax.fori_loop(0, n_rows, body, 0)


def _strided(start, stride, total, body):
    n_my = (total - start + stride - 1) // stride

    def loop_body(i, carry):
        body(start + i * stride)
        return carry

    lax.fori_loop(0, n_my, loop_body, 0)


def _zero_acc(acc, sid, zbuf):
    _fill_rows(zbuf, DCH, jnp.zeros((H,), jnp.float32))
    _strided(sid, NSC, NDC,
             lambda c: pltpu.sync_copy(zbuf.at[pl.ds(0, DCH)],
                                       acc.at[pl.ds(c * DCH, DCH)]))
    plsc.subcore_barrier()


def _dump_acc(acc, cid, sid, out):
    plsc.subcore_barrier()
    _strided(sid, NSC, NDC,
             lambda c: pltpu.sync_copy(acc.at[pl.ds(c * DCH, DCH)],
                                       out.at[cid, pl.ds(c * DCH, DCH)]))


@functools.partial(
    pl.kernel,
    out_type=jax.ShapeDtypeStruct((2, N_NODES, H), jnp.float32),
    mesh=_mesh,
    compiler_params=pltpu.CompilerParams(use_tc_tiling_on_sc=False),
    scratch_types=[
        pltpu.VMEM_SHARED((N_NODES, H), jnp.float32),
        pltpu.VMEM((CR, LANES), jnp.int32),
        pltpu.VMEM((CE, H), jnp.float32),
    ],
)
def _sc_count(eidx, out, acc, idxb, ones):
    cid = lax.axis_index("c")
    sid = lax.axis_index("s")
    _zero_acc(acc, sid, ones)
    _fill_rows(ones, CE, jnp.ones((H,), jnp.float32))
    plsc.subcore_barrier()

    def chunk(c):
        pltpu.sync_copy(eidx.at[cid, pl.ds(c * CR, CR)], idxb)
        for j in range(CR):
            pltpu.sync_copy(ones.at[pl.ds(j * LANES, LANES)],
                            acc.at[idxb.at[j]], add=True)

    _strided(sid, NSC, NCH, chunk)

    @pl.when(sid == NSC - 1)
    def _():
        pltpu.sync_copy(eidx.at[cid, pl.ds(NCH * CR, TAIL_R)],
                        idxb.at[pl.ds(0, TAIL_R)])
        for j in range(TAIL_R):
            pltpu.sync_copy(ones.at[pl.ds(j * LANES, LANES)],
                            acc.at[idxb.at[j]], add=True)

    _dump_acc(acc, cid, sid, out)


@functools.partial(
    pl.kernel,
    out_type=jax.ShapeDtypeStruct((2, N_NODES, H), jnp.float32),
    mesh=_mesh,
    compiler_params=pltpu.CompilerParams(use_tc_tiling_on_sc=False),
    scratch_types=[
        pltpu.VMEM_SHARED((N_NODES, H), jnp.float32),
        pltpu.VMEM((CR, LANES), jnp.int32),
        pltpu.VMEM((CE, H), jnp.float32),
    ],
)
def _sc_scatter(eidx, vals, out, acc, idxb, vbuf):
    cid = lax.axis_index("c")
    sid = lax.axis_index("s")
    _zero_acc(acc, sid, vbuf)

    def chunk(c):
        pltpu.sync_copy(eidx.at[cid, pl.ds(c * CR, CR)], idxb)
        pltpu.sync_copy(vals.at[pl.ds(c * CE, CE)], vbuf)
        for j in range(CR):
            pltpu.sync_copy(vbuf.at[pl.ds(j * LANES, LANES)],
                            acc.at[idxb.at[j]], add=True)

    _strided(sid, NSC, NCH, chunk)

    @pl.when(sid == NSC - 1)
    def _():
        pltpu.sync_copy(eidx.at[cid, pl.ds(NCH * CR, TAIL_R)],
                        idxb.at[pl.ds(0, TAIL_R)])
        pltpu.sync_copy(vals.at[pl.ds(NCH * CE, TAIL_R * LANES)],
                        vbuf.at[pl.ds(0, TAIL_R * LANES)])
        for j in range(TAIL_R):
            pltpu.sync_copy(vbuf.at[pl.ds(j * LANES, LANES)],
                            acc.at[idxb.at[j]], add=True)

    _dump_acc(acc, cid, sid, out)


@functools.partial(
    pl.kernel,
    out_type=[jax.ShapeDtypeStruct((E, H), jnp.float32),
              jax.ShapeDtypeStruct((E, H), jnp.float32)],
    mesh=_mesh,
    compiler_params=pltpu.CompilerParams(use_tc_tiling_on_sc=False),
    scratch_types=[
        pltpu.VMEM_SHARED((N_NODES, H), jnp.float32),
        pltpu.VMEM((CR, LANES), jnp.int32),
        pltpu.VMEM((CE, H), jnp.float32),
    ],
)
def _sc_gather(eidx, a_tab, b_tab, ga, gb, tab_s, idxb, buf):
    cid = lax.axis_index("c")
    sid = lax.axis_index("s")

    def prefetch(tab):
        _strided(sid, NSC, NDC,
                 lambda c: pltpu.sync_copy(tab.at[pl.ds(c * DCH, DCH)],
                                           tab_s.at[pl.ds(c * DCH, DCH)]))

    def do_rows(out, c, n_rows):
        pltpu.sync_copy(eidx.at[cid, pl.ds(c * CR, n_rows)],
                        idxb.at[pl.ds(0, n_rows)])
        for j in range(n_rows):
            pltpu.sync_copy(tab_s.at[idxb.at[j]],
                            buf.at[pl.ds(j * LANES, LANES)])
        pltpu.sync_copy(buf.at[pl.ds(0, n_rows * LANES)],
                        out.at[pl.ds(c * CE, n_rows * LANES)])

    def run(tab, out):
        prefetch(tab)
        plsc.subcore_barrier()
        _strided(sid, NSC, NCH, lambda c: do_rows(out, c, CR))

        @pl.when(sid == NSC - 1)
        def _():
            do_rows(out, NCH, TAIL_R)

    @pl.when(cid == 0)
    def _():
        run(a_tab, ga)

    @pl.when(cid == 1)
    def _():
        run(b_tab, gb)


BN = 1000
NB = N_NODES // BN
BE = 2000
NEB = (E // 8) // BE


def _graph_body(s_ref, c_ref, batch_ref, gsum_ref, gcnt_ref):
    i = pl.program_id(0)
    bt = batch_ref[0, 0, :]
    onehot = (bt[:, None] == lax.broadcasted_iota(jnp.int32, (BN, N_GRAPHS), 1)
              ).astype(jnp.float32)
    dims = (((0,), (0,)), ((), ()))
    gs = lax.dot_general(onehot, s_ref[...], dims,
                         preferred_element_type=jnp.float32)
    gc = lax.dot_general(onehot, c_ref[...], dims,
                         preferred_element_type=jnp.float32)

    @pl.when(i == 0)
    def _():
        gsum_ref[...] = jnp.zeros_like(gsum_ref)
        gcnt_ref[...] = jnp.zeros_like(gcnt_ref)

    gsum_ref[...] += gs
    gcnt_ref[...] += gc


def _tc_graph(S, cnt, batch3):
    return pl.pallas_call(
        _graph_body,
        grid=(NB,),
        in_specs=[
            pl.BlockSpec((BN, H), lambda i: (i, 0)),
            pl.BlockSpec((BN, H), lambda i: (i, 0)),
            pl.BlockSpec((1, 1, BN), lambda i: (i, 0, 0)),
        ],
        out_specs=[
            pl.BlockSpec((N_GRAPHS, H), lambda i: (0, 0)),
            pl.BlockSpec((N_GRAPHS, H), lambda i: (0, 0)),
        ],
        out_shape=[
            jax.ShapeDtypeStruct((N_GRAPHS, H), jnp.float32),
            jax.ShapeDtypeStruct((N_GRAPHS, H), jnp.float32),
        ],
    )(S, cnt, batch3)


def _tables_body(ss_ref, sd_ref, cs_ref, cd_ref, batch_ref, gsum_ref, gcnt_ref,
                 w_ref, b_ref, a_ref, b_out_ref):
    g = gsum_ref[...] / jnp.maximum(gcnt_ref[...], 1.0)
    C = jnp.dot(g, w_ref[5], preferred_element_type=jnp.float32) + b_ref[...]
    m_s = ss_ref[...] / jnp.maximum(cs_ref[...], 1.0)
    m_d = sd_ref[...] / jnp.maximum(cd_ref[...], 1.0)
    bt = batch_ref[0, 0, :]
    onehot = (bt[:, None] == lax.broadcasted_iota(jnp.int32, (BN, N_GRAPHS), 1)
              ).astype(jnp.float32)
    a_ref[...] = (jnp.dot(m_s, w_ref[1], preferred_element_type=jnp.float32)
                  + jnp.dot(m_d, w_ref[3], preferred_element_type=jnp.float32)
                  + jnp.dot(onehot, C, preferred_element_type=jnp.float32))
    b_out_ref[...] = (jnp.dot(m_s, w_ref[2], preferred_element_type=jnp.float32)
                      + jnp.dot(m_d, w_ref[4], preferred_element_type=jnp.float32))


def _tc_tables(Ss, Sd, cs, cd, batch3, gsum, gcnt, W, b):
    return pl.pallas_call(
        _tables_body,
        grid=(NB,),
        in_specs=[
            pl.BlockSpec((BN, H), lambda i: (i, 0)),
            pl.BlockSpec((BN, H), lambda i: (i, 0)),
            pl.BlockSpec((BN, H), lambda i: (i, 0)),
            pl.BlockSpec((BN, H), lambda i: (i, 0)),
            pl.BlockSpec((1, 1, BN), lambda i: (i, 0, 0)),
            pl.BlockSpec((N_GRAPHS, H), lambda i: (0, 0)),
            pl.BlockSpec((N_GRAPHS, H), lambda i: (0, 0)),
            pl.BlockSpec((6, H, H), lambda i: (0, 0, 0)),
            pl.BlockSpec((1, H), lambda i: (0, 0)),
        ],
        out_specs=[
            pl.BlockSpec((BN, H), lambda i: (i, 0)),
            pl.BlockSpec((BN, H), lambda i: (i, 0)),
        ],
        out_shape=[
            jax.ShapeDtypeStruct((N_NODES, H), jnp.float32),
            jax.ShapeDtypeStruct((N_NODES, H), jnp.float32),
        ],
    )(Ss, Sd, cs, cd, batch3, gsum, gcnt, W, b)


NR32 = E // 32
BR = 1000
NRB = NR32 // BR


def _in_body(in32_ref, mcat_ref, pcat_ref, xw_ref, in16_ref):
    v = in32_ref[...]
    vr = jnp.broadcast_to(v[:, None, :], (BR, 4, LANES)).reshape(4 * BR, LANES)
    row_q = lax.broadcasted_iota(jnp.int32, (4 * BR, LANES), 0) % 4
    col_g = lax.broadcasted_iota(jnp.int32, (4 * BR, LANES), 1) // 32
    xsel = jnp.where(row_q == col_g, vr, 0.0)
    xw_ref[...] = jnp.dot(xsel, mcat_ref[...],
                          preferred_element_type=jnp.float32)
    in16_ref[...] = jnp.dot(xsel, pcat_ref[...],
                            preferred_element_type=jnp.float32)


def _tc_in(in32, Mcat, Pcat):
    return pl.pallas_call(
        _in_body,
        grid=(NRB,),
        in_specs=[
            pl.BlockSpec((BR, LANES), lambda i: (i, 0)),
            pl.BlockSpec((LANES, LANES), lambda i: (0, 0)),
            pl.BlockSpec((LANES, LANES), lambda i: (0, 0)),
        ],
        out_specs=[
            pl.BlockSpec((4 * BR, LANES), lambda i: (i, 0)),
            pl.BlockSpec((4 * BR, LANES), lambda i: (i, 0)),
        ],
        out_shape=[
            jax.ShapeDtypeStruct((EW, LANES), jnp.float32),
            jax.ShapeDtypeStruct((EW, LANES), jnp.float32),
        ],
    )(in32, Mcat, Pcat)


def _in_weights(W0):
    def cat(w):
        return jnp.tile(jnp.kron(jnp.eye(8, dtype=jnp.float32), w), (4, 1))
    return cat(W0), cat(jnp.eye(4, H, dtype=jnp.float32))


def _edge0_body(xw_ref, ga_ref, gb_ref, w1_ref, b1_ref, x0_ref, h1_ref):
    x0 = xw_ref[...] + ga_ref[...] + gb_ref[...]
    x0_ref[...] = x0
    h1_ref[...] = jax.nn.gelu(
        jnp.dot(x0, w1_ref[...], preferred_element_type=jnp.float32)
        + b1_ref[...])


def _tc_edge0(xw128, Ga128, Gb128, W1d, b1t):
    return pl.pallas_call(
        _edge0_body,
        grid=(NEB,),
        in_specs=[
            pl.BlockSpec((BE, LANES), lambda i: (i, 0)),
            pl.BlockSpec((BE, LANES), lambda i: (i, 0)),
            pl.BlockSpec((BE, LANES), lambda i: (i, 0)),
            pl.BlockSpec((LANES, LANES), lambda i: (0, 0)),
            pl.BlockSpec((1, LANES), lambda i: (0, 0)),
        ],
        out_specs=[
            pl.BlockSpec((BE, LANES), lambda i: (i, 0)),
            pl.BlockSpec((BE, LANES), lambda i: (i, 0)),
        ],
        out_shape=[
            jax.ShapeDtypeStruct((E // 8, LANES), jnp.float32),
            jax.ShapeDtypeStruct((E // 8, LANES), jnp.float32),
        ],
    )(xw128, Ga128, Gb128, W1d, b1t)


def _edge_mid_body(x_ref, h_ref, ga_ref, gb_ref, w20_ref, w1n_ref, b1n_ref,
                   x_out_ref, h_out_ref):
    xn = x_ref[...] + jax.nn.gelu(
        jnp.dot(h_ref[...], w20_ref[...], preferred_element_type=jnp.float32)
        + ga_ref[...] + gb_ref[...])
    x_out_ref[...] = xn
    h_out_ref[...] = jax.nn.gelu(
        jnp.dot(xn, w1n_ref[...], preferred_element_type=jnp.float32)
        + b1n_ref[...])


def _tc_edge_mid(x128, h128, Ga128, Gb128, W20d, W1nd, b1nt):
    return pl.pallas_call(
        _edge_mid_body,
        grid=(NEB,),
        in_specs=[
            pl.BlockSpec((BE, LANES), lambda i: (i, 0)),
            pl.BlockSpec((BE, LANES), lambda i: (i, 0)),
            pl.BlockSpec((BE, LANES), lambda i: (i, 0)),
            pl.BlockSpec((BE, LANES), lambda i: (i, 0)),
            pl.BlockSpec((LANES, LANES), lambda i: (0, 0)),
            pl.BlockSpec((LANES, LANES), lambda i: (0, 0)),
            pl.BlockSpec((1, LANES), lambda i: (0, 0)),
        ],
        out_specs=[
            pl.BlockSpec((BE, LANES), lambda i: (i, 0)),
            pl.BlockSpec((BE, LANES), lambda i: (i, 0)),
        ],
        out_shape=[
            jax.ShapeDtypeStruct((E // 8, LANES), jnp.float32),
            jax.ShapeDtypeStruct((E // 8, LANES), jnp.float32),
        ],
    )(x128, h128, Ga128, Gb128, W20d, W1nd, b1nt)


def _edge_last_body(x_ref, h_ref, ga_ref, gb_ref, w20_ref, x_out_ref):
    x_out_ref[...] = x_ref[...] + jax.nn.gelu(
        jnp.dot(h_ref[...], w20_ref[...], preferred_element_type=jnp.float32)
        + ga_ref[...] + gb_ref[...])


def _tc_edge_last(x128, h128, Ga128, Gb128, W20d):
    return pl.pallas_call(
        _edge_last_body,
        grid=(NEB,),
        in_specs=[
            pl.BlockSpec((BE, LANES), lambda i: (i, 0)),
            pl.BlockSpec((BE, LANES), lambda i: (i, 0)),
            pl.BlockSpec((BE, LANES), lambda i: (i, 0)),
            pl.BlockSpec((BE, LANES), lambda i: (i, 0)),
            pl.BlockSpec((LANES, LANES), lambda i: (0, 0)),
        ],
        out_specs=pl.BlockSpec((BE, LANES), lambda i: (i, 0)),
        out_shape=jax.ShapeDtypeStruct((E // 8, LANES), jnp.float32),
    )(x128, h128, Ga128, Gb128, W20d)


def _out_body(gsum_ref, gcnt_ref, w_ref, b_ref, out_ref):
    g = gsum_ref[...] / jnp.maximum(gcnt_ref[...], 1.0)
    out_ref[...] = jnp.dot(g, w_ref[...],
                           preferred_element_type=jnp.float32) + b_ref[...]


def _tc_out(gsum, gcnt, Wpad, bpad):
    return pl.pallas_call(
        _out_body,
        in_specs=[
            pl.BlockSpec((N_GRAPHS, H), lambda: (0, 0)),
            pl.BlockSpec((N_GRAPHS, H), lambda: (0, 0)),
            pl.BlockSpec((H, H), lambda: (0, 0)),
            pl.BlockSpec((1, H), lambda: (0, 0)),
        ],
        out_specs=pl.BlockSpec((N_GRAPHS, H), lambda: (0, 0)),
        out_shape=jax.ShapeDtypeStruct((N_GRAPHS, H), jnp.float32),
    )(gsum, gcnt, Wpad, bpad)


def _bd(W):
    return jnp.kron(jnp.eye(8, dtype=W.dtype), W)


def kernel(in_rank2, edge_index, batch, W_in, b_in, W1_0, b1_0, W2_0, b2_0,
           W1_1, b1_1, W2_1, b2_1, W_out, b_out):
    eidx = edge_index.reshape(2, E_ROWS, LANES)
    batch3 = batch.reshape(NB, 1, BN)
    Mcat, Pcat = _in_weights(W_in[0])
    xw0, in16 = _tc_in(in_rank2.reshape(NR32, LANES), Mcat, Pcat)
    W_in16 = jnp.pad(W_in, ((0, 0), (0, H - W_in.shape[1]), (0, 0)))

    cnts = _sc_count(eidx)
    cs, cd = cnts[0], cnts[1]

    def round_tables(vals16, W6, b):
        S = _sc_scatter(eidx, vals16)
        Ss, Sd = S[0], S[1]
        gsum, gcnt = _tc_graph(Ss, cs, batch3)
        A, B = _tc_tables(Ss, Sd, cs, cd, batch3, gsum, gcnt,
                          W6, b.reshape(1, H))
        Ga, Gb = _sc_gather(eidx, A, B)
        return Ga.reshape(EW, LANES), Gb.reshape(EW, LANES), gcnt

    Ga0, Gb0, gcnt = round_tables(in16.reshape(E, H), W_in16, b_in)
    x0, h1 = _tc_edge0(xw0, Ga0, Gb0, _bd(W1_0),
                       jnp.tile(b1_0, 8).reshape(1, LANES))

    Ga1, Gb1, _ = round_tables(h1.reshape(E, H), W2_0, b2_0)
    x1, h2 = _tc_edge_mid(x0, h1, Ga1, Gb1, _bd(W2_0[0]), _bd(W1_1),
                          jnp.tile(b1_1, 8).reshape(1, LANES))

    Ga2, Gb2, _ = round_tables(h2.reshape(E, H), W2_1, b2_1)
    x2 = _tc_edge_last(x1, h2, Ga2, Gb2, _bd(W2_1[0]))

    S3 = _sc_scatter(eidx, x2.reshape(E, H))
    gsum3, _ = _tc_graph(S3[0], cs, batch3)
    Wo = jnp.pad(W_out, ((0, 0), (0, H - W_out.shape[1])))
    bo = jnp.pad(b_out, (0, H - b_out.shape[0])).reshape(1, H)
    out_full = _tc_out(gsum3, gcnt, Wo, bo)
    return out_full[:, :1]


# speedup vs baseline: 1.5056x; 1.4438x over previous
import functools

import jax
import jax.numpy as jnp
from jax import lax
from jax.experimental import pallas as pl
from jax.experimental.pallas import tpu as pltpu
from jax.experimental.pallas import tpu_sc as plsc

N_NODES = 100000
N_GRAPHS = 64
E = 1600000
H = 16

LANES = 128
E_ROWS = E // LANES
CR = 8
NCH = E_ROWS // CR
TAIL_R = E_ROWS - NCH * CR
CE = CR * LANES
NSC = 16
DCH = 1000
NDC = N_NODES // DCH
CW = CE // 8
EW = E // 8
TW = TAIL_R * LANES // 8

_mesh = plsc.VectorSubcoreMesh(core_axis_name="c", subcore_axis_name="s")


def _fill_rows(ref, n_rows, vec):
    def body(i, carry):
        ref[i] = vec
        return carry
    lax.fori_loop(0, n_rows, body, 0)


def _strided(start, stride, total, body):
    n_my = (total - start + stride - 1) // stride

    def loop_body(i, carry):
        body(start + i * stride)
        return carry

    lax.fori_loop(0, n_my, loop_body, 0)


def _zero_acc(acc, sid, zbuf):
    _fill_rows(zbuf, DCH, jnp.zeros((H,), jnp.float32))
    _strided(sid, NSC, NDC,
             lambda c: pltpu.sync_copy(zbuf.at[pl.ds(0, DCH)],
                                       acc.at[pl.ds(c * DCH, DCH)]))
    plsc.subcore_barrier()


def _dump_acc(acc, cid, sid, out):
    plsc.subcore_barrier()
    _strided(sid, NSC, NDC,
             lambda c: pltpu.sync_copy(acc.at[pl.ds(c * DCH, DCH)],
                                       out.at[cid, pl.ds(c * DCH, DCH)]))


@functools.partial(
    pl.kernel,
    out_type=jax.ShapeDtypeStruct((2, N_NODES, H), jnp.float32),
    mesh=_mesh,
    compiler_params=pltpu.CompilerParams(use_tc_tiling_on_sc=False),
    scratch_types=[
        pltpu.VMEM_SHARED((N_NODES, H), jnp.float32),
        pltpu.VMEM((CR, LANES), jnp.int32),
        pltpu.VMEM((CE, H), jnp.float32),
    ],
)
def _sc_count(eidx, out, acc, idxb, ones):
    cid = lax.axis_index("c")
    sid = lax.axis_index("s")
    _zero_acc(acc, sid, ones)
    _fill_rows(ones, CE, jnp.ones((H,), jnp.float32))
    plsc.subcore_barrier()

    def chunk(c):
        pltpu.sync_copy(eidx.at[cid, pl.ds(c * CR, CR)], idxb)
        for j in range(CR):
            pltpu.sync_copy(ones.at[pl.ds(j * LANES, LANES)],
                            acc.at[idxb.at[j]], add=True)

    _strided(sid, NSC, NCH, chunk)

    @pl.when(sid == NSC - 1)
    def _():
        pltpu.sync_copy(eidx.at[cid, pl.ds(NCH * CR, TAIL_R)],
                        idxb.at[pl.ds(0, TAIL_R)])
        for j in range(TAIL_R):
            pltpu.sync_copy(ones.at[pl.ds(j * LANES, LANES)],
                            acc.at[idxb.at[j]], add=True)

    _dump_acc(acc, cid, sid, out)


@functools.partial(
    pl.kernel,
    out_type=jax.ShapeDtypeStruct((2, N_NODES, H), jnp.float32),
    mesh=_mesh,
    compiler_params=pltpu.CompilerParams(use_tc_tiling_on_sc=False),
    scratch_types=[
        pltpu.VMEM_SHARED((N_NODES, H), jnp.float32),
        pltpu.VMEM((CR, LANES), jnp.int32),
        pltpu.VMEM((CE, H), jnp.float32),
    ],
)
def _sc_scatter(eidx, vals, out, acc, idxb, vbuf):
    cid = lax.axis_index("c")
    sid = lax.axis_index("s")
    _zero_acc(acc, sid, vbuf)

    def chunk(c):
        pltpu.sync_copy(eidx.at[cid, pl.ds(c * CR, CR)], idxb)
        pltpu.sync_copy(vals.at[pl.ds(c * CE, CE)], vbuf)
        for j in range(CR):
            pltpu.sync_copy(vbuf.at[pl.ds(j * LANES, LANES)],
                            acc.at[idxb.at[j]], add=True)

    _strided(sid, NSC, NCH, chunk)

    @pl.when(sid == NSC - 1)
    def _():
        pltpu.sync_copy(eidx.at[cid, pl.ds(NCH * CR, TAIL_R)],
                        idxb.at[pl.ds(0, TAIL_R)])
        pltpu.sync_copy(vals.at[pl.ds(NCH * CE, TAIL_R * LANES)],
                        vbuf.at[pl.ds(0, TAIL_R * LANES)])
        for j in range(TAIL_R):
            pltpu.sync_copy(vbuf.at[pl.ds(j * LANES, LANES)],
                            acc.at[idxb.at[j]], add=True)

    _dump_acc(acc, cid, sid, out)


@functools.partial(
    pl.kernel,
    out_type=[jax.ShapeDtypeStruct((E, H), jnp.float32),
              jax.ShapeDtypeStruct((E, H), jnp.float32)],
    mesh=_mesh,
    compiler_params=pltpu.CompilerParams(use_tc_tiling_on_sc=False),
    scratch_types=[
        pltpu.VMEM_SHARED((N_NODES, H), jnp.float32),
        pltpu.VMEM((CR, LANES), jnp.int32),
        pltpu.VMEM((CE, H), jnp.float32),
    ],
)
def _sc_gather(eidx, a_tab, b_tab, ga, gb, tab_s, idxb, buf):
    cid = lax.axis_index("c")
    sid = lax.axis_index("s")

    def prefetch(tab):
        _strided(sid, NSC, NDC,
                 lambda c: pltpu.sync_copy(tab.at[pl.ds(c * DCH, DCH)],
                                           tab_s.at[pl.ds(c * DCH, DCH)]))

    def do_rows(out, c, n_rows):
        pltpu.sync_copy(eidx.at[cid, pl.ds(c * CR, n_rows)],
                        idxb.at[pl.ds(0, n_rows)])
        for j in range(n_rows):
            pltpu.sync_copy(tab_s.at[idxb.at[j]],
                            buf.at[pl.ds(j * LANES, LANES)])
        pltpu.sync_copy(buf.at[pl.ds(0, n_rows * LANES)],
                        out.at[pl.ds(c * CE, n_rows * LANES)])

    def run(tab, out):
        prefetch(tab)
        plsc.subcore_barrier()
        _strided(sid, NSC, NCH, lambda c: do_rows(out, c, CR))

        @pl.when(sid == NSC - 1)
        def _():
            do_rows(out, NCH, TAIL_R)

    @pl.when(cid == 0)
    def _():
        run(a_tab, ga)

    @pl.when(cid == 1)
    def _():
        run(b_tab, gb)


BN = 1000
NB = N_NODES // BN
BE = 2000
NEB = (E // 8) // BE


def _graph_body(s_ref, c_ref, batch_ref, gsum_ref, gcnt_ref):
    i = pl.program_id(0)
    bt = batch_ref[0, 0, :]
    onehot = (bt[:, None] == lax.broadcasted_iota(jnp.int32, (BN, N_GRAPHS), 1)
              ).astype(jnp.float32)
    dims = (((0,), (0,)), ((), ()))
    gs = lax.dot_general(onehot, s_ref[...], dims,
                         preferred_element_type=jnp.float32)
    gc = lax.dot_general(onehot, c_ref[...], dims,
                         preferred_element_type=jnp.float32)

    @pl.when(i == 0)
    def _():
        gsum_ref[...] = jnp.zeros_like(gsum_ref)
        gcnt_ref[...] = jnp.zeros_like(gcnt_ref)

    gsum_ref[...] += gs
    gcnt_ref[...] += gc


def _tc_graph(S, cnt, batch3):
    return pl.pallas_call(
        _graph_body,
        grid=(NB,),
        in_specs=[
            pl.BlockSpec((BN, H), lambda i: (i, 0)),
            pl.BlockSpec((BN, H), lambda i: (i, 0)),
            pl.BlockSpec((1, 1, BN), lambda i: (i, 0, 0)),
        ],
        out_specs=[
            pl.BlockSpec((N_GRAPHS, H), lambda i: (0, 0)),
            pl.BlockSpec((N_GRAPHS, H), lambda i: (0, 0)),
        ],
        out_shape=[
            jax.ShapeDtypeStruct((N_GRAPHS, H), jnp.float32),
            jax.ShapeDtypeStruct((N_GRAPHS, H), jnp.float32),
        ],
    )(S, cnt, batch3)


def _tables_body(ss_ref, sd_ref, cs_ref, cd_ref, batch_ref, gsum_ref, gcnt_ref,
                 w_ref, b_ref, a_ref, b_out_ref):
    g = gsum_ref[...] / jnp.maximum(gcnt_ref[...], 1.0)
    C = jnp.dot(g, w_ref[5], preferred_element_type=jnp.float32) + b_ref[...]
    m_s = ss_ref[...] / jnp.maximum(cs_ref[...], 1.0)
    m_d = sd_ref[...] / jnp.maximum(cd_ref[...], 1.0)
    bt = batch_ref[0, 0, :]
    onehot = (bt[:, None] == lax.broadcasted_iota(jnp.int32, (BN, N_GRAPHS), 1)
              ).astype(jnp.float32)
    a_ref[...] = (jnp.dot(m_s, w_ref[1], preferred_element_type=jnp.float32)
                  + jnp.dot(m_d, w_ref[3], preferred_element_type=jnp.float32)
                  + jnp.dot(onehot, C, preferred_element_type=jnp.float32))
    b_out_ref[...] = (jnp.dot(m_s, w_ref[2], preferred_element_type=jnp.float32)
                      + jnp.dot(m_d, w_ref[4], preferred_element_type=jnp.float32))


def _tc_tables(Ss, Sd, cs, cd, batch3, gsum, gcnt, W, b):
    return pl.pallas_call(
        _tables_body,
        grid=(NB,),
        in_specs=[
            pl.BlockSpec((BN, H), lambda i: (i, 0)),
            pl.BlockSpec((BN, H), lambda i: (i, 0)),
            pl.BlockSpec((BN, H), lambda i: (i, 0)),
            pl.BlockSpec((BN, H), lambda i: (i, 0)),
            pl.BlockSpec((1, 1, BN), lambda i: (i, 0, 0)),
            pl.BlockSpec((N_GRAPHS, H), lambda i: (0, 0)),
            pl.BlockSpec((N_GRAPHS, H), lambda i: (0, 0)),
            pl.BlockSpec((6, H, H), lambda i: (0, 0, 0)),
            pl.BlockSpec((1, H), lambda i: (0, 0)),
        ],
        out_specs=[
            pl.BlockSpec((BN, H), lambda i: (i, 0)),
            pl.BlockSpec((BN, H), lambda i: (i, 0)),
        ],
        out_shape=[
            jax.ShapeDtypeStruct((N_NODES, H), jnp.float32),
            jax.ShapeDtypeStruct((N_NODES, H), jnp.float32),
        ],
    )(Ss, Sd, cs, cd, batch3, gsum, gcnt, W, b)


NR32 = E // 32
TB = 250
NRB = (E // LANES) // TB


def _in_body(in4_ref, mcat_ref, pcat_ref, xw_ref, in16_ref):
    v = in4_ref[...].reshape(TB, 4, LANES)
    row_b = lax.broadcasted_iota(jnp.int32, (16 * TB, LANES), 0) % 16
    col_g = lax.broadcasted_iota(jnp.int32, (16 * TB, LANES), 1) // 8
    msk = row_b == col_g
    planes = []
    for f in range(4):
        vf = jnp.broadcast_to(v[:, f, :][:, None, :],
                              (TB, 16, LANES)).reshape(16 * TB, LANES)
        planes.append(jnp.where(msk, vf, 0.0))
    X = jnp.concatenate(planes, axis=1)
    xw_ref[...] = jnp.dot(X, mcat_ref[...],
                          preferred_element_type=jnp.float32)
    in16_ref[...] = jnp.dot(X, pcat_ref[...],
                            preferred_element_type=jnp.float32)


def _tc_in(in4, Mcat, Pcat):
    return pl.pallas_call(
        _in_body,
        grid=(NRB,),
        in_specs=[
            pl.BlockSpec((4 * TB, LANES), lambda i: (i, 0)),
            pl.BlockSpec((4 * LANES, LANES), lambda i: (0, 0)),
            pl.BlockSpec((4 * LANES, LANES), lambda i: (0, 0)),
        ],
        out_specs=[
            pl.BlockSpec((16 * TB, LANES), lambda i: (i, 0)),
            pl.BlockSpec((16 * TB, LANES), lambda i: (i, 0)),
        ],
        out_shape=[
            jax.ShapeDtypeStruct((EW, LANES), jnp.float32),
            jax.ShapeDtypeStruct((EW, LANES), jnp.float32),
        ],
    )(in4, Mcat, Pcat)


def _in_weights(W0):
    def cat(w):
        return jnp.concatenate(
            [jnp.tile(jnp.kron(jnp.eye(8, dtype=jnp.float32), w[f:f + 1, :]),
                      (16, 1)) for f in range(4)], axis=0)
    return cat(W0), cat(jnp.eye(4, H, dtype=jnp.float32))


def _edge0_body(xw_ref, ga_ref, gb_ref, w1_ref, b1_ref, x0_ref, h1_ref):
    x0 = xw_ref[...] + ga_ref[...] + gb_ref[...]
    x0_ref[...] = x0
    h1_ref[...] = jax.nn.gelu(
        jnp.dot(x0, w1_ref[...], preferred_element_type=jnp.float32)
        + b1_ref[...])


def _tc_edge0(xw128, Ga128, Gb128, W1d, b1t):
    return pl.pallas_call(
        _edge0_body,
        grid=(NEB,),
        in_specs=[
            pl.BlockSpec((BE, LANES), lambda i: (i, 0)),
            pl.BlockSpec((BE, LANES), lambda i: (i, 0)),
            pl.BlockSpec((BE, LANES), lambda i: (i, 0)),
            pl.BlockSpec((LANES, LANES), lambda i: (0, 0)),
            pl.BlockSpec((1, LANES), lambda i: (0, 0)),
        ],
        out_specs=[
            pl.BlockSpec((BE, LANES), lambda i: (i, 0)),
            pl.BlockSpec((BE, LANES), lambda i: (i, 0)),
        ],
        out_shape=[
            jax.ShapeDtypeStruct((E // 8, LANES), jnp.float32),
            jax.ShapeDtypeStruct((E // 8, LANES), jnp.float32),
        ],
    )(xw128, Ga128, Gb128, W1d, b1t)


def _edge_mid_body(x_ref, h_ref, ga_ref, gb_ref, w20_ref, w1n_ref, b1n_ref,
                   x_out_ref, h_out_ref):
    xn = x_ref[...] + jax.nn.gelu(
        jnp.dot(h_ref[...], w20_ref[...], preferred_element_type=jnp.float32)
        + ga_ref[...] + gb_ref[...])
    x_out_ref[...] = xn
    h_out_ref[...] = jax.nn.gelu(
        jnp.dot(xn, w1n_ref[...], preferred_element_type=jnp.float32)
        + b1n_ref[...])


def _tc_edge_mid(x128, h128, Ga128, Gb128, W20d, W1nd, b1nt):
    return pl.pallas_call(
        _edge_mid_body,
        grid=(NEB,),
        in_specs=[
            pl.BlockSpec((BE, LANES), lambda i: (i, 0)),
            pl.BlockSpec((BE, LANES), lambda i: (i, 0)),
            pl.BlockSpec((BE, LANES), lambda i: (i, 0)),
            pl.BlockSpec((BE, LANES), lambda i: (i, 0)),
            pl.BlockSpec((LANES, LANES), lambda i: (0, 0)),
            pl.BlockSpec((LANES, LANES), lambda i: (0, 0)),
            pl.BlockSpec((1, LANES), lambda i: (0, 0)),
        ],
        out_specs=[
            pl.BlockSpec((BE, LANES), lambda i: (i, 0)),
            pl.BlockSpec((BE, LANES), lambda i: (i, 0)),
        ],
        out_shape=[
            jax.ShapeDtypeStruct((E // 8, LANES), jnp.float32),
            jax.ShapeDtypeStruct((E // 8, LANES), jnp.float32),
        ],
    )(x128, h128, Ga128, Gb128, W20d, W1nd, b1nt)


def _edge_last_body(x_ref, h_ref, ga_ref, gb_ref, w20_ref, x_out_ref):
    x_out_ref[...] = x_ref[...] + jax.nn.gelu(
        jnp.dot(h_ref[...], w20_ref[...], preferred_element_type=jnp.float32)
        + ga_ref[...] + gb_ref[...])


def _tc_edge_last(x128, h128, Ga128, Gb128, W20d):
    return pl.pallas_call(
        _edge_last_body,
        grid=(NEB,),
        in_specs=[
            pl.BlockSpec((BE, LANES), lambda i: (i, 0)),
            pl.BlockSpec((BE, LANES), lambda i: (i, 0)),
            pl.BlockSpec((BE, LANES), lambda i: (i, 0)),
            pl.BlockSpec((BE, LANES), lambda i: (i, 0)),
            pl.BlockSpec((LANES, LANES), lambda i: (0, 0)),
        ],
        out_specs=pl.BlockSpec((BE, LANES), lambda i: (i, 0)),
        out_shape=jax.ShapeDtypeStruct((E // 8, LANES), jnp.float32),
    )(x128, h128, Ga128, Gb128, W20d)


def _out_body(gsum_ref, gcnt_ref, w_ref, b_ref, out_ref):
    g = gsum_ref[...] / jnp.maximum(gcnt_ref[...], 1.0)
    out_ref[...] = jnp.dot(g, w_ref[...],
                           preferred_element_type=jnp.float32) + b_ref[...]


def _tc_out(gsum, gcnt, Wpad, bpad):
    return pl.pallas_call(
        _out_body,
        in_specs=[
            pl.BlockSpec((N_GRAPHS, H), lambda: (0, 0)),
            pl.BlockSpec((N_GRAPHS, H), lambda: (0, 0)),
            pl.BlockSpec((H, H), lambda: (0, 0)),
            pl.BlockSpec((1, H), lambda: (0, 0)),
        ],
        out_specs=pl.BlockSpec((N_GRAPHS, H), lambda: (0, 0)),
        out_shape=jax.ShapeDtypeStruct((N_GRAPHS, H), jnp.float32),
    )(gsum, gcnt, Wpad, bpad)


def _bd(W):
    return jnp.kron(jnp.eye(8, dtype=W.dtype), W)


def kernel(in_rank2, edge_index, batch, W_in, b_in, W1_0, b1_0, W2_0, b2_0,
           W1_1, b1_1, W2_1, b2_1, W_out, b_out):
    eidx = edge_index.reshape(2, E_ROWS, LANES)
    batch3 = batch.reshape(NB, 1, BN)
    Mcat, Pcat = _in_weights(W_in[0])
    in4 = jnp.swapaxes(in_rank2.reshape(E // LANES, LANES, 4), 1, 2)
    xw0, in16 = _tc_in(in4.reshape(NR32, LANES), Mcat, Pcat)
    W_in16 = jnp.pad(W_in, ((0, 0), (0, H - W_in.shape[1]), (0, 0)))

    cnts = _sc_count(eidx)
    cs, cd = cnts[0], cnts[1]

    def round_tables(vals16, W6, b):
        S = _sc_scatter(eidx, vals16)
        Ss, Sd = S[0], S[1]
        gsum, gcnt = _tc_graph(Ss, cs, batch3)
        A, B = _tc_tables(Ss, Sd, cs, cd, batch3, gsum, gcnt,
                          W6, b.reshape(1, H))
        Ga, Gb = _sc_gather(eidx, A, B)
        return Ga.reshape(EW, LANES), Gb.reshape(EW, LANES), gcnt

    Ga0, Gb0, gcnt = round_tables(in16.reshape(E, H), W_in16, b_in)
    x0, h1 = _tc_edge0(xw0, Ga0, Gb0, _bd(W1_0),
                       jnp.tile(b1_0, 8).reshape(1, LANES))

    Ga1, Gb1, _ = round_tables(h1.reshape(E, H), W2_0, b2_0)
    x1, h2 = _tc_edge_mid(x0, h1, Ga1, Gb1, _bd(W2_0[0]), _bd(W1_1),
                          jnp.tile(b1_1, 8).reshape(1, LANES))

    Ga2, Gb2, _ = round_tables(h2.reshape(E, H), W2_1, b2_1)
    x2 = _tc_edge_last(x1, h2, Ga2, Gb2, _bd(W2_1[0]))

    S3 = _sc_scatter(eidx, x2.reshape(E, H))
    gsum3, _ = _tc_graph(S3[0], cs, batch3)
    Wo = jnp.pad(W_out, ((0, 0), (0, H - W_out.shape[1])))
    bo = jnp.pad(b_out, (0, H - b_out.shape[0])).reshape(1, H)
    out_full = _tc_out(gsum3, gcnt, Wo, bo)
    return out_full[:, :1]
